# baseline (speedup 1.0000x reference)
import functools

import jax
import jax.numpy as jnp
from jax import lax
from jax.experimental import pallas as pl
from jax.experimental.pallas import tpu as pltpu
from jax.experimental.pallas import tpu_sc as plsc

_ROWS = 8192
_DIM = 1024
_NW = 32
_RPW = _ROWS // _NW
_SC_CHUNK = 32
_NCH = _RPW // _SC_CHUNK


@functools.partial(
    pl.kernel,
    mesh=plsc.VectorSubcoreMesh(core_axis_name="c", subcore_axis_name="s"),
    out_type=jax.ShapeDtypeStruct((_ROWS, _DIM), jnp.float32),
    scratch_types=[
        pltpu.VMEM((2, _SC_CHUNK, _DIM), jnp.float32),
        pltpu.SemaphoreType.DMA((2,)),
        pltpu.SemaphoreType.DMA((2,)),
    ],
)
def _sc_copy(emb_hbm, out_hbm, buf, in_sems, out_sems):
    wid = lax.axis_index("s") * 2 + lax.axis_index("c")
    base = wid * _RPW

    def in_copy(g, slot):
        return pltpu.make_async_copy(
            emb_hbm.at[pl.ds(base + g * _SC_CHUNK, _SC_CHUNK), :],
            buf.at[slot],
            in_sems.at[slot],
        )

    def out_copy(g, slot):
        return pltpu.make_async_copy(
            buf.at[slot],
            out_hbm.at[pl.ds(base + g * _SC_CHUNK, _SC_CHUNK), :],
            out_sems.at[slot],
        )

    in_copy(0, 0).start()
    for g in range(_NCH):
        slot = g % 2
        in_copy(g, slot).wait()
        out_copy(g, slot).start()
        if g + 1 < _NCH:
            nslot = (g + 1) % 2
            if g >= 1:
                out_copy(g - 1, nslot).wait()
            in_copy(g + 1, nslot).start()
    out_copy(_NCH - 2, (_NCH - 2) % 2).wait()
    out_copy(_NCH - 1, (_NCH - 1) % 2).wait()


def kernel(x, emb):
    del x
    return _sc_copy(emb)
